# baseline (speedup 1.0000x reference)
import functools

import jax
import jax.numpy as jnp
from jax import lax
from jax.experimental import pallas as pl
from jax.experimental.pallas import tpu as pltpu
from jax.experimental.pallas import tpu_sc as plsc

_NC = 2
_NS = 16
_NW = _NC * _NS
_L = 16

_R, _C = 200, 16384
_CW = _C // _NW
_CCH = 128
_NCH = _CW // _CCH
_RH = 104
_HALVES = ((0, _RH), (_RH, _R - _RH))
_TBL = 128


@functools.partial(
    pl.kernel,
    mesh=plsc.VectorSubcoreMesh(core_axis_name="c", subcore_axis_name="s"),
    out_type=jax.ShapeDtypeStruct((_R, _C), jnp.int32),
    scratch_types=[
        pltpu.VMEM((_TBL,), jnp.int32),
        pltpu.VMEM((_R, _CCH), jnp.int32),
        pltpu.VMEM((_R, _CCH), jnp.int32),
        pltpu.VMEM((_R, _CCH), jnp.int32),
        pltpu.VMEM((_R, _CCH), jnp.int32),
        pltpu.SemaphoreType.DMA,
        pltpu.SemaphoreType.DMA,
        pltpu.SemaphoreType.DMA,
        pltpu.SemaphoreType.DMA,
        pltpu.SemaphoreType.DMA,
        pltpu.SemaphoreType.DMA,
        pltpu.SemaphoreType.DMA,
        pltpu.SemaphoreType.DMA,
        pltpu.SemaphoreType.DMA,
    ],
    compiler_params=pltpu.CompilerParams(needs_layout_passes=False),
)
def _sc_convert(st_hbm, table_hbm, out_hbm, table_v, in_v0, in_v1,
                out_v0, out_v1, in_sem00, in_sem01, in_sem10, in_sem11,
                out_sem00, out_sem01, out_sem10, out_sem11, tbl_sem):
    in_bufs = (in_v0, in_v1)
    out_bufs = (out_v0, out_v1)
    in_sems = ((in_sem00, in_sem01), (in_sem10, in_sem11))
    out_sems = ((out_sem00, out_sem01), (out_sem10, out_sem11))
    wid = lax.axis_index("s") * _NC + lax.axis_index("c")
    col0 = wid * _CW

    col_vecs = [jnp.full((_L,), g * _L, jnp.int32) + lax.iota(jnp.int32, _L)
                for g in range(_CCH // _L)]

    def start_in(ci):
        slot = ci % 2
        return [
            pltpu.async_copy(
                st_hbm.at[pl.ds(r0, nr), pl.ds(col0 + ci * _CCH, _CCH)],
                in_bufs[slot].at[pl.ds(r0, nr), pl.ds(0, _CCH)],
                in_sems[slot][h])
            for h, (r0, nr) in enumerate(_HALVES)
        ]

    in_copies = [None] * _NCH
    out_copies = [None] * _NCH
    in_copies[0] = start_in(0)
    in_copies[1] = start_in(1)
    tbl_copy = pltpu.async_copy(table_hbm, table_v.at[pl.ds(0, 120)], tbl_sem)
    tbl_copy.wait()

    for ci in range(_NCH):
        slot = ci % 2
        if ci >= 2:
            for cp in out_copies[ci - 2]:
                cp.wait()

        in_b = in_bufs[slot]
        out_b = out_bufs[slot]
        out_copies[ci] = []

        for h, (r0, nr) in enumerate(_HALVES):
            in_copies[ci][h].wait()

            @plsc.parallel_loop(r0, r0 + nr, 1, unroll=2)
            def _gather(r):
                row_vec = jnp.full((_L,), r, jnp.int32)
                for cv in col_vecs:
                    idx = plsc.load_gather(in_b, [row_vec, cv])
                    vals = plsc.load_gather(table_v, [idx])
                    plsc.store_scatter(out_b, [row_vec, cv], vals)

            out_copies[ci].append(pltpu.async_copy(
                out_b.at[pl.ds(r0, nr), pl.ds(0, _CCH)],
                out_hbm.at[pl.ds(r0, nr), pl.ds(col0 + ci * _CCH, _CCH)],
                out_sems[slot][h]))

        if ci + 2 < _NCH:
            in_copies[ci + 2] = start_in(ci + 2)

    for ci in (_NCH - 2, _NCH - 1):
        for cp in out_copies[ci]:
            cp.wait()


def kernel(species, conv_tensor):
    out_t = _sc_convert(species.T, conv_tensor)
    return out_t.T


# speedup vs baseline: 1.0426x; 1.0426x over previous
import functools

import jax
import jax.numpy as jnp
from jax import lax
from jax.experimental import pallas as pl
from jax.experimental.pallas import tpu as pltpu
from jax.experimental.pallas import tpu_sc as plsc

_NC = 2
_NS = 16
_NW = _NC * _NS
_L = 16

_R, _C = 200, 16384
_CW = _C // _NW
_CCH = 128
_NCH = _CW // _CCH
_TBL = 128


@functools.partial(
    pl.kernel,
    mesh=plsc.VectorSubcoreMesh(core_axis_name="c", subcore_axis_name="s"),
    out_type=jax.ShapeDtypeStruct((_R, _C), jnp.int32),
    scratch_types=[
        pltpu.VMEM((_TBL,), jnp.int32),
        pltpu.VMEM((_R, _CCH), jnp.int32),
        pltpu.VMEM((_R, _CCH), jnp.int32),
        pltpu.VMEM((_R, _CCH), jnp.int32),
        pltpu.VMEM((_R, _CCH), jnp.int32),
        pltpu.SemaphoreType.DMA,
        pltpu.SemaphoreType.DMA,
        pltpu.SemaphoreType.DMA,
        pltpu.SemaphoreType.DMA,
        pltpu.SemaphoreType.DMA,
    ],
    compiler_params=pltpu.CompilerParams(needs_layout_passes=False),
)
def _sc_convert(st_hbm, table_hbm, out_hbm, table_v, in_v0, in_v1,
                out_v0, out_v1, in_sem0, in_sem1, out_sem0, out_sem1,
                tbl_sem):
    in_bufs = (in_v0, in_v1)
    out_bufs = (out_v0, out_v1)
    in_sems = (in_sem0, in_sem1)
    out_sems = (out_sem0, out_sem1)
    wid = lax.axis_index("s") * _NC + lax.axis_index("c")
    col0 = wid * _CW

    col_vecs = [jnp.full((_L,), g * _L, jnp.int32) + lax.iota(jnp.int32, _L)
                for g in range(_CCH // _L)]

    in_copies = [None] * _NCH
    out_copies = [None] * _NCH
    in_copies[0] = pltpu.async_copy(
        st_hbm.at[pl.ds(0, _R), pl.ds(col0, _CCH)], in_bufs[0], in_sems[0])
    in_copies[1] = pltpu.async_copy(
        st_hbm.at[pl.ds(0, _R), pl.ds(col0 + _CCH, _CCH)],
        in_bufs[1], in_sems[1])
    tbl_copy = pltpu.async_copy(table_hbm, table_v.at[pl.ds(0, 120)], tbl_sem)
    tbl_copy.wait()

    for ci in range(_NCH):
        slot = ci % 2
        in_copies[ci].wait()
        if ci >= 2:
            out_copies[ci - 2].wait()

        in_b = in_bufs[slot]
        out_b = out_bufs[slot]

        @plsc.parallel_loop(0, _R, 1, unroll=2)
        def _gather(r):
            row_vec = jnp.full((_L,), r, jnp.int32)
            for cv in col_vecs:
                idx = plsc.load_gather(in_b, [row_vec, cv])
                vals = plsc.load_gather(table_v, [idx])
                plsc.store_scatter(out_b, [row_vec, cv], vals)

        if ci + 2 < _NCH:
            in_copies[ci + 2] = pltpu.async_copy(
                st_hbm.at[pl.ds(0, _R), pl.ds(col0 + (ci + 2) * _CCH, _CCH)],
                in_bufs[slot], in_sems[slot])
        out_copies[ci] = pltpu.async_copy(
            out_b, out_hbm.at[pl.ds(0, _R), pl.ds(col0 + ci * _CCH, _CCH)],
            out_sems[slot])

    out_copies[_NCH - 2].wait()
    out_copies[_NCH - 1].wait()


def kernel(species, conv_tensor):
    out_t = _sc_convert(species.T, conv_tensor)
    return out_t.T
